# Initial kernel scaffold; baseline (speedup 1.0000x reference)
import jax
import jax.numpy as jnp
from jax.experimental import pallas as pl


def kernel(x, edge_index, W1, b1, W2, b2, Wl, bl):
    raise NotImplementedError("write your pallas kernel here")


# speedup vs baseline: 10.4464x; 10.4464x over previous
import functools

import jax
import jax.numpy as jnp
from jax import lax
from jax.experimental import pallas as pl
from jax.experimental.pallas import tpu as pltpu
from jax.experimental.pallas import tpu_sc as plsc

N = 10000
NPAD = 10240
E = 320000
D_IN = 128
D_H = 256
NC = 2
NS = 16
K = 80
RB = 512
LANES = 16

_MESH = plsc.VectorSubcoreMesh(core_axis_name="c", subcore_axis_name="s")


@functools.partial(
    pl.kernel,
    mesh=_MESH,
    compiler_params=pltpu.CompilerParams(needs_layout_passes=False),
    out_type=jax.ShapeDtypeStruct((NC * NS, NPAD), jnp.float32),
    scratch_types=[
        pltpu.VMEM((NPAD,), jnp.float32),
        pltpu.VMEM((K,), jnp.int32),
    ],
)
def _sc_degree(dst_hbm, out_hbm, hist, idxb):
    c = lax.axis_index("c")
    s = lax.axis_index("s")
    wid = c * NS + s
    epw = E // (NC * NS)
    zero16 = jnp.zeros((LANES,), jnp.float32)
    one16 = jnp.ones((LANES,), jnp.float32)

    def zbody(i, carry):
        hist[pl.ds(i * LANES, LANES)] = zero16
        return carry

    lax.fori_loop(0, NPAD // LANES, zbody, 0)

    base = wid * epw

    def body(g, carry):
        pltpu.sync_copy(dst_hbm.at[pl.ds(base + g * K, K)], idxb)
        for j in range(K // LANES):
            iv = idxb[pl.ds(j * LANES, LANES)]
            plsc.addupdate_scatter(hist, [iv], one16)
        return carry

    lax.fori_loop(0, epw // K, body, 0)
    pltpu.sync_copy(hist, out_hbm.at[wid])


@functools.partial(
    pl.kernel,
    mesh=_MESH,
    compiler_params=pltpu.CompilerParams(needs_layout_passes=False),
    out_type=jax.ShapeDtypeStruct((NC, NPAD, D_IN), jnp.float32),
    scratch_types=[
        pltpu.VMEM((K,), jnp.int32),
        pltpu.VMEM((1, K), jnp.int32),
        pltpu.VMEM((K, D_IN), jnp.float32),
        pltpu.VMEM((40, D_IN), jnp.float32),
        pltpu.VMEM_SHARED((NPAD, D_IN), jnp.float32),
        pltpu.SemaphoreType.DMA,
    ],
)
def _sc_agg1(xs_hbm, src_hbm, dst_hbm, out_hbm, sidx, didx, rows, zbuf, acc, sem):
    c = lax.axis_index("c")
    s = lax.axis_index("s")
    epw = E // (NC * NS)
    zero16 = jnp.zeros((LANES,), jnp.float32)

    def zb(i, carry):
        for j in range(D_IN // LANES):
            zbuf[i, pl.ds(j * LANES, LANES)] = zero16
        return carry

    lax.fori_loop(0, 40, zb, 0)

    rpt = NPAD // NS

    def zc(q, carry):
        pltpu.sync_copy(zbuf, acc.at[pl.ds(s * rpt + q * 40, 40)])
        return carry

    lax.fori_loop(0, rpt // 40, zc, 0)
    plsc.subcore_barrier()

    base = (c * NS + s) * epw

    def body(g, carry):
        eb = base + g * K
        pltpu.sync_copy(src_hbm.at[pl.ds(eb, K)], sidx)
        pltpu.sync_copy(dst_hbm.at[pl.ds(eb, K)], didx.at[0])
        pltpu.async_copy(xs_hbm.at[sidx], rows, sem).wait()
        pltpu.sync_copy(rows, acc.at[didx.at[0]], add=True)
        return carry

    lax.fori_loop(0, epw // K, body, 0)
    plsc.subcore_barrier()
    pltpu.sync_copy(acc.at[pl.ds(s * rpt, rpt)], out_hbm.at[c, pl.ds(s * rpt, rpt)])


@functools.partial(
    pl.kernel,
    mesh=_MESH,
    compiler_params=pltpu.CompilerParams(needs_layout_passes=False),
    out_type=jax.ShapeDtypeStruct((NC, NPAD, D_IN), jnp.float32),
    scratch_types=[
        pltpu.VMEM((K,), jnp.int32),
        pltpu.VMEM((1, K), jnp.int32),
        pltpu.VMEM((K, D_IN), jnp.float32),
        pltpu.VMEM((40, D_IN), jnp.float32),
        pltpu.VMEM_SHARED((NPAD, D_IN), jnp.float32),
        pltpu.SemaphoreType.DMA,
    ],
)
def _sc_agg2(y_hbm, src_hbm, dst_hbm, out_hbm, sidx, didx, rows, zbuf, acc, sem):
    c = lax.axis_index("c")
    s = lax.axis_index("s")
    ept = E // NS
    zero16 = jnp.zeros((LANES,), jnp.float32)

    def zb(i, carry):
        for j in range(D_IN // LANES):
            zbuf[i, pl.ds(j * LANES, LANES)] = zero16
        return carry

    lax.fori_loop(0, 40, zb, 0)

    rpt = NPAD // NS

    def zc(q, carry):
        pltpu.sync_copy(zbuf, acc.at[pl.ds(s * rpt + q * 40, 40)])
        return carry

    lax.fori_loop(0, rpt // 40, zc, 0)
    plsc.subcore_barrier()

    base = s * ept
    off = c * NPAD

    def body(g, carry):
        eb = base + g * K
        pltpu.sync_copy(src_hbm.at[pl.ds(eb, K)], sidx)
        pltpu.sync_copy(dst_hbm.at[pl.ds(eb, K)], didx.at[0])
        for j in range(K // LANES):
            sidx[pl.ds(j * LANES, LANES)] = sidx[pl.ds(j * LANES, LANES)] + off
        pltpu.async_copy(y_hbm.at[sidx], rows, sem).wait()
        pltpu.sync_copy(rows, acc.at[didx.at[0]], add=True)
        return carry

    lax.fori_loop(0, ept // K, body, 0)
    plsc.subcore_barrier()
    pltpu.sync_copy(acc.at[pl.ds(s * rpt, rpt)], out_hbm.at[c, pl.ds(s * rpt, rpt)])


def _prep_body(degp_ref, x_ref, dinv_ref, xs_ref):
    i = pl.program_id(0)
    deg = jnp.sum(degp_ref[...], axis=1, keepdims=True) + 1.0
    dv = lax.rsqrt(jnp.maximum(deg, 1e-12))
    row = lax.broadcasted_iota(jnp.int32, (RB, 1), 0) + i * RB
    dv = jnp.where(row < N, dv, 0.0)
    dinv_ref[...] = dv
    xs_ref[...] = dv * x_ref[...]


def _mid_body(agg_ref, xs_ref, dinv_ref, w1_ref, b1_ref, y_ref):
    dv = dinv_ref[...]
    t = agg_ref[0] + agg_ref[1] + xs_ref[...]
    pre = dv * t
    h = jnp.dot(pre, w1_ref[...], preferred_element_type=jnp.float32, precision=lax.Precision.HIGHEST)
    h = jnp.maximum(h + b1_ref[...], 0.0)
    y = dv * h
    y_ref[0] = y[:, :D_IN]
    y_ref[1] = y[:, D_IN:]


def _out_body(agg2_ref, y3_ref, dinv_ref, w2_ref, b2_ref, wl_ref, bl_ref, o_ref):
    dv = dinv_ref[...]
    t0 = dv * (agg2_ref[0] + y3_ref[0])
    t1 = dv * (agg2_ref[1] + y3_ref[1])
    h = (jnp.dot(t0, w2_ref[0], preferred_element_type=jnp.float32, precision=lax.Precision.HIGHEST)
         + jnp.dot(t1, w2_ref[1], preferred_element_type=jnp.float32, precision=lax.Precision.HIGHEST))
    h = jnp.maximum(h + b2_ref[...], 0.0)
    o_ref[...] = jnp.dot(h, wl_ref[...], preferred_element_type=jnp.float32, precision=lax.Precision.HIGHEST) + bl_ref[...]


_GRID = (NPAD // RB,)


def _tc_prep(degp_t, x_pad):
    return pl.pallas_call(
        _prep_body,
        grid=_GRID,
        in_specs=[
            pl.BlockSpec((RB, NC * NS), lambda i: (i, 0)),
            pl.BlockSpec((RB, D_IN), lambda i: (i, 0)),
        ],
        out_specs=[
            pl.BlockSpec((RB, 1), lambda i: (i, 0)),
            pl.BlockSpec((RB, D_IN), lambda i: (i, 0)),
        ],
        out_shape=[
            jax.ShapeDtypeStruct((NPAD, 1), jnp.float32),
            jax.ShapeDtypeStruct((NPAD, D_IN), jnp.float32),
        ],
    )(degp_t, x_pad)


def _tc_mid(agg1, xs, dinv, w1, b1):
    return pl.pallas_call(
        _mid_body,
        grid=_GRID,
        in_specs=[
            pl.BlockSpec((NC, RB, D_IN), lambda i: (0, i, 0)),
            pl.BlockSpec((RB, D_IN), lambda i: (i, 0)),
            pl.BlockSpec((RB, 1), lambda i: (i, 0)),
            pl.BlockSpec((D_IN, D_H), lambda i: (0, 0)),
            pl.BlockSpec((1, D_H), lambda i: (0, 0)),
        ],
        out_specs=pl.BlockSpec((NC, RB, D_IN), lambda i: (0, i, 0)),
        out_shape=jax.ShapeDtypeStruct((NC, NPAD, D_IN), jnp.float32),
    )(agg1, xs, dinv, w1, b1)


def _tc_out(agg2, y3, dinv, w2, b2, wl, bl):
    return pl.pallas_call(
        _out_body,
        grid=_GRID,
        in_specs=[
            pl.BlockSpec((NC, RB, D_IN), lambda i: (0, i, 0)),
            pl.BlockSpec((NC, RB, D_IN), lambda i: (0, i, 0)),
            pl.BlockSpec((RB, 1), lambda i: (i, 0)),
            pl.BlockSpec((NC, D_IN, D_H), lambda i: (0, 0, 0)),
            pl.BlockSpec((1, D_H), lambda i: (0, 0)),
            pl.BlockSpec((D_H, 1), lambda i: (0, 0)),
            pl.BlockSpec((1, 1), lambda i: (0, 0)),
        ],
        out_specs=pl.BlockSpec((RB, 1), lambda i: (i, 0)),
        out_shape=jax.ShapeDtypeStruct((NPAD, 1), jnp.float32),
    )(agg2, y3, dinv, w2, b2, wl, bl)


def kernel(x, edge_index, W1, b1, W2, b2, Wl, bl):
    src = edge_index[0]
    dst = edge_index[1]
    x_pad = jnp.pad(x, ((0, NPAD - N), (0, 0)))

    degp = _sc_degree(dst)
    dinv, xs = _tc_prep(degp.T, x_pad)
    agg1 = _sc_agg1(xs, src, dst)
    y3 = _tc_mid(agg1, xs, dinv, W1, b1.reshape(1, D_H))
    y_flat = y3.reshape(NC * NPAD, D_IN)
    agg2 = _sc_agg2(y_flat, src, dst)
    out = _tc_out(agg2, y3, dinv, W2.reshape(NC, D_IN, D_H),
                  b2.reshape(1, D_H), Wl, bl.reshape(1, 1))
    return out[:N, 0]
